# baseline (speedup 1.0000x reference)
import functools

import jax
import jax.numpy as jnp
from jax import lax
from jax.experimental import pallas as pl
from jax.experimental.pallas import tpu as pltpu
from jax.experimental.pallas import tpu_sc as plsc

N_NODES = 10000
D = 128
BN_EPS = 1e-5

NC = 2
NS = 16
NW = NC * NS
CB = 128

N_PAD = 10112
ROWS_PER_TILE = N_PAD // NS
GARBAGE_ROW = N_NODES + 8


def _sc_aggregate(x, packed):
    n_chunks = packed.shape[1]
    mesh = plsc.VectorSubcoreMesh(core_axis_name="c", subcore_axis_name="s")

    @functools.partial(
        pl.kernel,
        mesh=mesh,
        out_type=jax.ShapeDtypeStruct((NC, N_PAD, D), jnp.float32),
        scratch_types=[
            pltpu.VMEM((n_chunks, CB), jnp.int32),
            pltpu.VMEM((1, CB), jnp.int32),
            pltpu.VMEM((1, CB), jnp.int32),
            pltpu.VMEM((1, CB), jnp.int32),
            pltpu.VMEM((1, CB), jnp.int32),
            pltpu.VMEM((CB, D), jnp.float32),
            pltpu.VMEM((CB, D), jnp.float32),
            pltpu.VMEM_SHARED((N_PAD, D), jnp.float32),
            pltpu.SemaphoreType.DMA,
            pltpu.SemaphoreType.DMA,
        ],
    )
    def body(x_hbm, packed_hbm, out_hbm,
             pk_v, src0, src1, dst0, dst1, rows0, rows1, agg_sh, sem0, sem1):
        c = lax.axis_index("c")
        s = lax.axis_index("s")
        w = c * NS + s
        pltpu.sync_copy(packed_hbm.at[w], pk_v)

        zvec = jnp.zeros((16,), jnp.float32)
        nvd = D // 16

        def zstore(k, carry):
            rows0[k // nvd, pl.ds((k % nvd) * 16, 16)] = zvec
            return carry

        lax.fori_loop(0, CB * nvd, zstore, 0)
        base = s * ROWS_PER_TILE
        for r0 in range(0, ROWS_PER_TILE, CB):
            nr = min(CB, ROWS_PER_TILE - r0)
            pltpu.sync_copy(rows0.at[pl.ds(0, nr)],
                            agg_sh.at[pl.ds(base + r0, nr)])
        plsc.subcore_barrier()

        def unpack(j, src_c, dst_c):
            for t in range(CB // 16):
                v = pk_v[j, pl.ds(t * 16, 16)]
                src_c[0, pl.ds(t * 16, 16)] = lax.bitwise_and(v, 0x3FFF)
                dst_c[0, pl.ds(t * 16, 16)] = lax.shift_right_logical(v, 14)

        def gather(src_c, buf, sem):
            return pltpu.async_copy(x_hbm.at[src_c.at[0]], buf, sem)

        def gwait(src_c, buf, sem):
            pltpu.make_async_copy(x_hbm.at[src_c.at[0]], buf, sem).wait()

        def scatter(dst_c, buf):
            pltpu.sync_copy(buf, agg_sh.at[dst_c.at[0]], add=True)

        unpack(0, src0, dst0)
        gather(src0, rows0, sem0)

        def two_chunks(i, carry):
            j = 2 * i
            unpack(j + 1, src1, dst1)
            gather(src1, rows1, sem1)
            gwait(src0, rows0, sem0)
            scatter(dst0, rows0)

            @pl.when(j + 2 < n_chunks)
            def _():
                unpack(j + 2, src0, dst0)
                gather(src0, rows0, sem0)

            gwait(src1, rows1, sem1)
            scatter(dst1, rows1)
            return carry

        lax.fori_loop(0, n_chunks // 2, two_chunks, 0)
        plsc.subcore_barrier()
        pltpu.sync_copy(agg_sh.at[pl.ds(s * ROWS_PER_TILE, ROWS_PER_TILE)],
                        out_hbm.at[c, pl.ds(s * ROWS_PER_TILE, ROWS_PER_TILE)])

    return body(x, packed)


BLK = 1000
N_BLKS = N_NODES // BLK


def _mlp1_body(p_ref, x_ref, eps_ref, w1_ref, b1_ref, h_ref, stats_ref):
    i = pl.program_id(0)
    out = p_ref[0] + p_ref[1] + (1.0 + eps_ref[0]) * x_ref[...]
    h = lax.dot_general(out, w1_ref[...], (((1,), (1,)), ((), ())),
                        preferred_element_type=jnp.float32) + b1_ref[...]
    h_ref[...] = h

    @pl.when(i == 0)
    def _():
        stats_ref[...] = jnp.zeros_like(stats_ref)

    stats_ref[0:1, :] += jnp.sum(h, axis=0, keepdims=True)
    stats_ref[1:2, :] += jnp.sum(h * h, axis=0, keepdims=True)


def _mlp2_body(h_ref, stats_ref, gamma_ref, beta_ref, w2_ref, b2_ref, y_ref):
    inv_n = 1.0 / N_NODES
    mean = stats_ref[0:1, :] * inv_n
    var = stats_ref[1:2, :] * inv_n - mean * mean
    inv = lax.rsqrt(var + BN_EPS)
    hn = (h_ref[...] - mean) * (inv * gamma_ref[...]) + beta_ref[...]
    hr = jnp.maximum(hn, 0.0)
    y_ref[...] = lax.dot_general(hr, w2_ref[...], (((1,), (1,)), ((), ())),
                                 preferred_element_type=jnp.float32) + b2_ref[...]


def kernel(x, edge_index, eps, W1, b1, gamma, beta, W2, b2):
    src = edge_index[0].astype(jnp.int32)
    dst = edge_index[1].astype(jnp.int32)
    e = src.shape[0]
    per_w = -(-e // (NW * 2 * CB)) * 2 * CB
    e_pad = per_w * NW
    src_p = jnp.concatenate([src, jnp.zeros((e_pad - e,), jnp.int32)])
    dst_p = jnp.concatenate(
        [dst, jnp.full((e_pad - e,), GARBAGE_ROW, jnp.int32)])
    packed = (src_p | (dst_p << 14)).reshape(NW, per_w // CB, CB)

    partials = _sc_aggregate(x, packed)

    h, stats = pl.pallas_call(
        _mlp1_body,
        grid=(N_BLKS,),
        in_specs=[
            pl.BlockSpec((NC, BLK, D), lambda i: (0, i, 0)),
            pl.BlockSpec((BLK, D), lambda i: (i, 0)),
            pl.BlockSpec(memory_space=pltpu.MemorySpace.SMEM),
            pl.BlockSpec((D, D), lambda i: (0, 0)),
            pl.BlockSpec((1, D), lambda i: (0, 0)),
        ],
        out_specs=[
            pl.BlockSpec((BLK, D), lambda i: (i, 0)),
            pl.BlockSpec((8, D), lambda i: (0, 0)),
        ],
        out_shape=[
            jax.ShapeDtypeStruct((N_NODES, D), jnp.float32),
            jax.ShapeDtypeStruct((8, D), jnp.float32),
        ],
    )(partials, x, eps, W1, b1.reshape(1, D))

    y = pl.pallas_call(
        _mlp2_body,
        grid=(N_BLKS,),
        in_specs=[
            pl.BlockSpec((BLK, D), lambda i: (i, 0)),
            pl.BlockSpec((8, D), lambda i: (0, 0)),
            pl.BlockSpec((1, D), lambda i: (0, 0)),
            pl.BlockSpec((1, D), lambda i: (0, 0)),
            pl.BlockSpec((D, D), lambda i: (0, 0)),
            pl.BlockSpec((1, D), lambda i: (0, 0)),
        ],
        out_specs=pl.BlockSpec((BLK, D), lambda i: (i, 0)),
        out_shape=jax.ShapeDtypeStruct((N_NODES, D), jnp.float32),
    )(h, stats, gamma.reshape(1, D), beta.reshape(1, D), W2,
      b2.reshape(1, D))

    return y
